# Initial kernel scaffold; baseline (speedup 1.0000x reference)
import jax
import jax.numpy as jnp
from jax.experimental import pallas as pl


def kernel(x, median, num_track):
    raise NotImplementedError("write your pallas kernel here")


# speedup vs baseline: 16.8731x; 16.8731x over previous
import jax
import jax.numpy as jnp
from jax.experimental import pallas as pl
from jax.experimental.pallas import tpu as pltpu

_N = 32768
_C = 768
_BC = 128
_RB = 256
_K = _N // 2
_BR_SUB = 2048


def _key_to_float(k):
    i = k ^ (jax.lax.shift_right_arithmetic(k, 31) & jnp.int32(0x7FFFFFFF))
    return jax.lax.bitcast_convert_type(i, jnp.float32)


def _median_body(x_ref, med_ref, nt_ref, newmed_ref):
    def outer(b, pk):
        q = pk + jnp.left_shift(jnp.int32(1), 31 - b)
        qf = _key_to_float(q)

        def inner(r, acc):
            chunk = x_ref[pl.ds(r * _RB, _RB), :]
            return acc + jnp.sum((chunk < qf).astype(jnp.int32), axis=0,
                                 keepdims=True)

        cnt = jax.lax.fori_loop(0, _N // _RB, inner,
                                jnp.zeros((1, _BC), jnp.int32))
        return jnp.where(cnt <= _K, q, pk)

    pk0 = jnp.full((1, _BC), jnp.iinfo(jnp.int32).min, jnp.int32)
    pk = jax.lax.fori_loop(0, 32, outer, pk0)

    med = _key_to_float(pk)
    nt = nt_ref[0, 0]
    newmed_ref[...] = (med_ref[...] * nt + med) / (nt + 1.0)


def _sub_body(x_ref, newmed_ref, o_ref):
    o_ref[...] = x_ref[...] - newmed_ref[...]


@jax.jit
def _mean_shift(x, median, nt):
    new_med = pl.pallas_call(
        _median_body,
        grid=(_C // _BC,),
        in_specs=[
            pl.BlockSpec((_N, _BC), lambda j: (0, j)),
            pl.BlockSpec((1, _BC), lambda j: (0, j)),
            pl.BlockSpec((1, 1), lambda j: (0, 0), memory_space=pltpu.SMEM),
        ],
        out_specs=pl.BlockSpec((1, _BC), lambda j: (0, j)),
        out_shape=jax.ShapeDtypeStruct((1, _C), jnp.float32),
        compiler_params=pltpu.CompilerParams(
            dimension_semantics=("arbitrary",),
        ),
    )(x, median, nt)

    return pl.pallas_call(
        _sub_body,
        grid=(_N // _BR_SUB,),
        in_specs=[
            pl.BlockSpec((_BR_SUB, _C), lambda i: (i, 0)),
            pl.BlockSpec((1, _C), lambda i: (0, 0)),
        ],
        out_specs=pl.BlockSpec((_BR_SUB, _C), lambda i: (i, 0)),
        out_shape=jax.ShapeDtypeStruct((_N, _C), jnp.float32),
        compiler_params=pltpu.CompilerParams(
            dimension_semantics=("arbitrary",),
        ),
    )(x, new_med)


def kernel(x, median, num_track):
    nt = num_track.astype(jnp.float32).reshape(1, 1)
    return _mean_shift(x, median, nt)


# speedup vs baseline: 21.2027x; 1.2566x over previous
import jax
import jax.numpy as jnp
from jax.experimental import pallas as pl
from jax.experimental.pallas import tpu as pltpu

_N = 32768
_C = 768
_BC = 128
_RB = 256
_K = _N // 2
_BR_SUB = 2048


def _key_to_float(k):
    i = k ^ (jax.lax.shift_right_arithmetic(k, 31) & jnp.int32(0x7FFFFFFF))
    return jax.lax.bitcast_convert_type(i, jnp.float32)


def _median_body(x_ref, med_ref, nt_ref, newmed_ref):
    def outer(b, pk):
        q = pk + jnp.left_shift(jnp.int32(1), 31 - b)
        qf = _key_to_float(q)

        def inner(r, acc8):
            chunk = x_ref[pl.ds(r * _RB, _RB), :]
            m = (chunk < qf).astype(jnp.int32)
            return acc8 + jnp.sum(m.reshape(_RB // 8, 8, _BC), axis=0)

        acc8 = jax.lax.fori_loop(0, _N // _RB, inner,
                                 jnp.zeros((8, _BC), jnp.int32))
        cnt = jnp.sum(acc8, axis=0, keepdims=True)
        return jnp.where(cnt <= _K, q, pk)

    pk0 = jnp.full((1, _BC), jnp.iinfo(jnp.int32).min, jnp.int32)
    pk = jax.lax.fori_loop(0, 32, outer, pk0)

    med = _key_to_float(pk)
    nt = nt_ref[0, 0]
    newmed_ref[...] = (med_ref[...] * nt + med) / (nt + 1.0)


def _sub_body(x_ref, newmed_ref, o_ref):
    o_ref[...] = x_ref[...] - newmed_ref[...]


@jax.jit
def _mean_shift(x, median, nt):
    new_med = pl.pallas_call(
        _median_body,
        grid=(_C // _BC,),
        in_specs=[
            pl.BlockSpec((_N, _BC), lambda j: (0, j)),
            pl.BlockSpec((1, _BC), lambda j: (0, j)),
            pl.BlockSpec((1, 1), lambda j: (0, 0), memory_space=pltpu.SMEM),
        ],
        out_specs=pl.BlockSpec((1, _BC), lambda j: (0, j)),
        out_shape=jax.ShapeDtypeStruct((1, _C), jnp.float32),
        compiler_params=pltpu.CompilerParams(
            dimension_semantics=("arbitrary",),
        ),
    )(x, median, nt)

    return pl.pallas_call(
        _sub_body,
        grid=(_N // _BR_SUB,),
        in_specs=[
            pl.BlockSpec((_BR_SUB, _C), lambda i: (i, 0)),
            pl.BlockSpec((1, _C), lambda i: (0, 0)),
        ],
        out_specs=pl.BlockSpec((_BR_SUB, _C), lambda i: (i, 0)),
        out_shape=jax.ShapeDtypeStruct((_N, _C), jnp.float32),
        compiler_params=pltpu.CompilerParams(
            dimension_semantics=("arbitrary",),
        ),
    )(x, new_med)


def kernel(x, median, num_track):
    nt = num_track.astype(jnp.float32).reshape(1, 1)
    return _mean_shift(x, median, nt)
